# baseline (speedup 1.0000x reference)
import functools

import jax
import jax.numpy as jnp
from jax import lax
from jax.experimental import pallas as pl
from jax.experimental.pallas import tpu as pltpu
from jax.experimental.pallas import tpu_sc as plsc

_T, _B, _V_GEN, _V_EXT = 32, 64, 10000, 10500
_N = _T * _B
_NC, _NS, _L = 2, 16, 16
_NW = _NC * _NS
_PW = _N // _NW
_CH = _PW // _L


def _sc_body(ptr_hbm, gen_hbm, ct_hbm, gt_hbm, out_hbm,
             ct_v, gt_v, pidx_v, gidx_v, pv_v, gv_v, sv_v, sem):
    wid = lax.axis_index("s") * _NC + lax.axis_index("c")
    base = wid * _PW
    pltpu.sync_copy(ct_hbm.at[pl.ds(base, _PW)], ct_v)
    pltpu.sync_copy(gt_hbm.at[pl.ds(base, _PW)], gt_v)
    lane = lax.iota(jnp.int32, 16)
    for j in range(_CH):
        n = base + j * _L + lane
        t = lax.rem(n, _T)
        b = lax.div(n, _T)
        row = t * _B + b
        pidx_v[pl.ds(j * _L, _L)] = row * _V_EXT + ct_v[pl.ds(j * _L, _L)]
        gidx_v[pl.ds(j * _L, _L)] = row * _V_GEN + gt_v[pl.ds(j * _L, _L)]
    cp = pltpu.async_copy(ptr_hbm.at[pidx_v], pv_v, sem)
    cg = pltpu.async_copy(gen_hbm.at[gidx_v], gv_v, sem)
    cp.wait()
    cg.wait()
    for j in range(_CH):
        sl = pl.ds(j * _L, _L)
        eq = ct_v[sl] == gt_v[sl]
        sv_v[sl] = pv_v[sl] + jnp.where(eq, gv_v[sl], jnp.float32(0.0))
    pltpu.sync_copy(sv_v, out_hbm.at[pl.ds(base, _PW)])


_sc_gather = functools.partial(
    pl.kernel,
    out_type=jax.ShapeDtypeStruct((_N,), jnp.float32),
    mesh=plsc.VectorSubcoreMesh(core_axis_name="c", subcore_axis_name="s"),
    scratch_types=[
        pltpu.VMEM((_PW,), jnp.int32),
        pltpu.VMEM((_PW,), jnp.int32),
        pltpu.VMEM((_PW,), jnp.int32),
        pltpu.VMEM((_PW,), jnp.int32),
        pltpu.VMEM((_PW,), jnp.float32),
        pltpu.VMEM((_PW,), jnp.float32),
        pltpu.VMEM((_PW,), jnp.float32),
        pltpu.SemaphoreType.DMA,
    ],
)(_sc_body)


def _tc_body(s_ref, m_ref, o_ref):
    valid = jnp.float32(1.0) - m_ref[...]
    lp = jnp.log(s_ref[...]) * valid
    o_ref[0, 0] = -jnp.sum(lp) / jnp.sum(valid)


_tc_loss = pl.pallas_call(
    _tc_body,
    out_shape=jax.ShapeDtypeStruct((1, 1), jnp.float32),
    in_specs=[
        pl.BlockSpec(memory_space=pltpu.VMEM),
        pl.BlockSpec(memory_space=pltpu.VMEM),
    ],
    out_specs=pl.BlockSpec(memory_space=pltpu.SMEM),
)


def kernel(pointer_probability, generator_probability, copy_targets,
           target_output_features, target_mask):
    ptr_flat = pointer_probability.reshape(-1)
    gen_flat = generator_probability.reshape(-1)
    ct = copy_targets.astype(jnp.int32).reshape(-1)
    gt = target_output_features.astype(jnp.int32).reshape(-1)
    s = _sc_gather(ptr_flat, gen_flat, ct, gt)
    m = target_mask.astype(jnp.float32).reshape(_L, _N // _L)
    out = _tc_loss(s.reshape(_L, _N // _L), m)
    return out[0, 0]


# speedup vs baseline: 7.3548x; 7.3548x over previous
import functools

import jax
import jax.numpy as jnp
from jax import lax
from jax.experimental import pallas as pl
from jax.experimental.pallas import tpu as pltpu
from jax.experimental.pallas import tpu_sc as plsc

_T, _B, _V_GEN, _V_EXT = 32, 64, 10000, 10500
_N = _T * _B
_NC, _NS, _L = 2, 16, 16
_NW = _NC * _NS
_PW = _N // _NW
_CH = _PW // _L


def _gather_one(src_hbm, idx_v, base, tile_v, out_v, sv_slot, lane, sem):
    copies = []
    for j in range(_CH):
        iv16 = idx_v[pl.ds(j * _L, _L)]
        for k in range(_L):
            n = base + j * _L + k
            t = lax.rem(n, _T)
            b = lax.div(n, _T)
            row = t * _B + b
            r0 = pl.multiple_of(lax.bitwise_and(row, ~7), 8)
            c0 = pl.multiple_of(lax.bitwise_and(iv16[k], ~127), 128)
            copies.append(pltpu.async_copy(
                src_hbm.at[pl.ds(r0, 8), pl.ds(c0, 128)],
                tile_v.at[j * _L + k], sem))
    for c in copies:
        c.wait()
    for j in range(_CH):
        sl = pl.ds(j * _L, _L)
        n16 = base + j * _L + lane
        t16 = lax.rem(n16, _T)
        b16 = lax.div(n16, _T)
        rmod = lax.bitwise_and(t16 * _B + b16, 7)
        cmod = lax.bitwise_and(idx_v[sl], 127)
        out_v[sl] = plsc.load_gather(tile_v, [j * _L + lane, rmod, cmod])


def _sc_body(ptr_hbm, gen_hbm, ct_hbm, gt_hbm, out_hbm,
             ct_v, gt_v, tile_v, pv_v, gv_v, sv_v, sem):
    wid = lax.axis_index("s") * _NC + lax.axis_index("c")
    base = wid * _PW
    pltpu.sync_copy(ct_hbm.at[pl.ds(base, _PW)], ct_v)
    pltpu.sync_copy(gt_hbm.at[pl.ds(base, _PW)], gt_v)
    lane = lax.iota(jnp.int32, 16)
    _gather_one(ptr_hbm, ct_v, base, tile_v, pv_v, 0, lane, sem)
    _gather_one(gen_hbm, gt_v, base, tile_v, gv_v, 0, lane, sem)
    for j in range(_CH):
        sl = pl.ds(j * _L, _L)
        eq = ct_v[sl] == gt_v[sl]
        sv_v[sl] = pv_v[sl] + jnp.where(eq, gv_v[sl], jnp.float32(0.0))
    pltpu.sync_copy(sv_v, out_hbm.at[pl.ds(base, _PW)])


_sc_gather = functools.partial(
    pl.kernel,
    out_type=jax.ShapeDtypeStruct((_N,), jnp.float32),
    mesh=plsc.VectorSubcoreMesh(core_axis_name="c", subcore_axis_name="s"),
    compiler_params=pltpu.CompilerParams(
        use_tc_tiling_on_sc=True, needs_layout_passes=False),
    scratch_types=[
        pltpu.VMEM((_PW,), jnp.int32),
        pltpu.VMEM((_PW,), jnp.int32),
        pltpu.VMEM((_PW, 8, 128), jnp.float32),
        pltpu.VMEM((_PW,), jnp.float32),
        pltpu.VMEM((_PW,), jnp.float32),
        pltpu.VMEM((_PW,), jnp.float32),
        pltpu.SemaphoreType.DMA,
    ],
)(_sc_body)


def _tc_body(s_ref, m_ref, o_ref):
    valid = jnp.float32(1.0) - m_ref[...]
    lp = jnp.log(s_ref[...]) * valid
    o_ref[0, 0] = -jnp.sum(lp) / jnp.sum(valid)


_tc_loss = pl.pallas_call(
    _tc_body,
    out_shape=jax.ShapeDtypeStruct((1, 1), jnp.float32),
    in_specs=[
        pl.BlockSpec(memory_space=pltpu.VMEM),
        pl.BlockSpec(memory_space=pltpu.VMEM),
    ],
    out_specs=pl.BlockSpec(memory_space=pltpu.SMEM),
)


def kernel(pointer_probability, generator_probability, copy_targets,
           target_output_features, target_mask):
    ptr2d = pointer_probability.reshape(_N, _V_EXT)
    gen2d = generator_probability.reshape(_N, _V_GEN)
    ct = copy_targets.astype(jnp.int32).reshape(-1)
    gt = target_output_features.astype(jnp.int32).reshape(-1)
    s = _sc_gather(ptr2d, gen2d, ct, gt)
    m = target_mask.astype(jnp.float32).reshape(_L, _N // _L)
    out = _tc_loss(s.reshape(_L, _N // _L), m)
    return out[0, 0]


# speedup vs baseline: 7.6732x; 1.0433x over previous
import functools

import jax
import jax.numpy as jnp
from jax import lax
from jax.experimental import pallas as pl
from jax.experimental.pallas import tpu as pltpu
from jax.experimental.pallas import tpu_sc as plsc

_T, _B, _V_GEN, _V_EXT = 32, 64, 10000, 10500
_N = _T * _B
_NC, _NS, _L = 2, 16, 16
_NW = _NC * _NS
_PW = _N // _NW
_CH = _PW // _L


def _window_src(src_hbm, wid, k, c):
    row_const = (k % _T) * _B + k // _T
    r0 = pl.multiple_of(lax.bitwise_and(row_const + 2 * wid, ~7), 8)
    c0 = pl.multiple_of(lax.bitwise_and(c, ~127), 128)
    return src_hbm.at[pl.ds(r0, 8), pl.ds(c0, 128)]


def _sc_body(ptr_hbm, gen_hbm, tgt_hbm, out_hbm,
             ct_v, gt_v, pw_v, gw_v, sv_v, sem):
    wid = lax.axis_index("s") * _NC + lax.axis_index("c")
    base = wid * _PW
    pltpu.sync_copy(tgt_hbm.at[pl.ds(base, _PW)], ct_v)
    pltpu.sync_copy(tgt_hbm.at[pl.ds(_N + base, _PW)], gt_v)
    lane = lax.iota(jnp.int32, 16)
    for h in range(2):
        copies = []
        for j in (2 * h, 2 * h + 1):
            ct16 = ct_v[pl.ds(j * _L, _L)]
            gt16 = gt_v[pl.ds(j * _L, _L)]
            for i in range(_L):
                k = j * _L + i
                slot = k - h * 2 * _L
                copies.append(pltpu.async_copy(
                    _window_src(ptr_hbm, wid, k, ct16[i]), pw_v.at[slot],
                    sem))

                @pl.when(ct16[i] == gt16[i])
                def _():
                    pltpu.async_copy(
                        _window_src(gen_hbm, wid, k, gt16[i]),
                        gw_v.at[slot], sem)
        for c in copies:
            c.wait()
        for j in (2 * h, 2 * h + 1):
            ct16 = ct_v[pl.ds(j * _L, _L)]
            gt16 = gt_v[pl.ds(j * _L, _L)]
            for i in range(_L):
                k = j * _L + i
                slot = k - h * 2 * _L

                @pl.when(ct16[i] == gt16[i])
                def _():
                    pltpu.make_async_copy(
                        _window_src(gen_hbm, wid, k, gt16[i]),
                        gw_v.at[slot], sem).wait()
        for j in (2 * h, 2 * h + 1):
            sl = pl.ds(j * _L, _L)
            row16 = ((j % 2) * _L + lane) * _B + j // 2 + 2 * wid
            rmod = lax.bitwise_and(row16, 7)
            slot16 = (j - 2 * h) * _L + lane
            p = plsc.load_gather(pw_v, [slot16, rmod,
                                        lax.bitwise_and(ct_v[sl], 127)])
            g = plsc.load_gather(gw_v, [slot16, rmod,
                                        lax.bitwise_and(gt_v[sl], 127)])
            eq = ct_v[sl] == gt_v[sl]
            sv_v[sl] = p + jnp.where(eq, g, jnp.float32(0.0))
    pltpu.sync_copy(sv_v, out_hbm.at[pl.ds(base, _PW)])


_sc_gather = functools.partial(
    pl.kernel,
    out_type=jax.ShapeDtypeStruct((_N,), jnp.float32),
    mesh=plsc.VectorSubcoreMesh(core_axis_name="c", subcore_axis_name="s"),
    compiler_params=pltpu.CompilerParams(
        use_tc_tiling_on_sc=True, needs_layout_passes=False),
    scratch_types=[
        pltpu.VMEM((_PW,), jnp.int32),
        pltpu.VMEM((_PW,), jnp.int32),
        pltpu.VMEM((2 * _L, 8, 128), jnp.float32),
        pltpu.VMEM((2 * _L, 8, 128), jnp.float32),
        pltpu.VMEM((_PW,), jnp.float32),
        pltpu.SemaphoreType.DMA,
    ],
)(_sc_body)


def _tc_body(s_ref, m_ref, o_ref):
    valid = jnp.float32(1.0) - m_ref[...]
    lp = jnp.log(s_ref[...]) * valid
    o_ref[0, 0] = -jnp.sum(lp) / jnp.sum(valid)


_tc_loss = pl.pallas_call(
    _tc_body,
    out_shape=jax.ShapeDtypeStruct((1, 1), jnp.float32),
    in_specs=[
        pl.BlockSpec(memory_space=pltpu.VMEM),
        pl.BlockSpec(memory_space=pltpu.VMEM),
    ],
    out_specs=pl.BlockSpec(memory_space=pltpu.SMEM),
)


def kernel(pointer_probability, generator_probability, copy_targets,
           target_output_features, target_mask):
    ptr2d = pointer_probability.reshape(_N, _V_EXT)
    gen2d = generator_probability.reshape(_N, _V_GEN)
    tgt = jnp.concatenate([
        copy_targets.astype(jnp.int32).reshape(-1),
        target_output_features.astype(jnp.int32).reshape(-1)])
    s = _sc_gather(ptr2d, gen2d, tgt)
    m = target_mask.astype(jnp.float32).reshape(-1)
    out = _tc_loss(s, m)
    return out[0, 0]
